# Initial kernel scaffold; baseline (speedup 1.0000x reference)
import jax
import jax.numpy as jnp
from jax.experimental import pallas as pl


def kernel(input_embedding, W_lin, b_lin, W_q, W_k, W_v, W_add, b_add, W_fin, b_fin):
    raise NotImplementedError("write your pallas kernel here")


# speedup vs baseline: 3.6727x; 3.6727x over previous
import functools
import math

import numpy as np
import jax
import jax.numpy as jnp
from jax.experimental import pallas as pl
from jax.experimental.pallas import tpu as pltpu

N = 2048
D_IN = 1024
D = 768
NCLS = 16
U = 40
SEL = 64
QT = 256
ET = 128
SCALE = 1.0 / math.sqrt(D)


@functools.lru_cache(maxsize=None)
def _sample_counts():
    with jax.ensure_compile_time_eval():
        idx = np.asarray(jax.random.randint(jax.random.key(42), (N, U), 0, N))
    c = np.zeros((N, N), dtype=np.int8)
    np.add.at(c, (np.arange(N)[:, None], idx.astype(np.int64)), 1)
    return c


def _dot_t(a, b):
    return jax.lax.dot_general(a, b, (((1,), (1,)), ((), ())),
                               preferred_element_type=jnp.float32)


def _proj_body(emb_ref, wl_ref, bl_ref, wq_ref, wk_ref, wv_ref, wa_ref, ba_ref,
               q_ref, k_ref, v_ref, add_ref):
    x = _dot_t(emb_ref[...], wl_ref[...]) + bl_ref[...]
    q_ref[...] = _dot_t(x, wq_ref[...])
    k_ref[...] = _dot_t(x, wk_ref[...])
    v_ref[...] = _dot_t(x, wv_ref[...])
    add_ref[...] = _dot_t(x, wa_ref[...]) + ba_ref[...]


def _measure_body(q_ref, k_ref, c_ref, m_ref):
    s = _dot_t(q_ref[...], k_ref[...])
    cf = c_ref[...].astype(jnp.float32)
    smax = jnp.max(jnp.where(cf > 0.5, s, -1e30), axis=1, keepdims=True)
    ssum = jnp.sum(s * cf, axis=1, keepdims=True)
    m_ref[...] = smax - ssum * (1.0 / N)


def _topk_body(m_ref, its_ref, itl_ref):
    m = m_ref[...]
    rows = jax.lax.broadcasted_iota(jnp.int32, (16, 128), 0)
    lanes = jax.lax.broadcasted_iota(jnp.int32, (16, 128), 1)
    gidx = rows * 128 + lanes
    sel_s = jax.lax.broadcasted_iota(jnp.int32, (SEL, 1), 0)
    sel_l = jax.lax.broadcasted_iota(jnp.int32, (1, SEL), 1)

    def step(j, carry):
        mm, acc_s, acc_l = carry
        cmax = jnp.max(mm)
        sel = jnp.min(jnp.where(mm == cmax, gidx, jnp.int32(1 << 30)))
        acc_s = jnp.where(sel_s == j, sel, acc_s)
        acc_l = jnp.where(sel_l == j, sel, acc_l)
        mm = jnp.where(gidx == sel, -3e38, mm)
        return mm, acc_s, acc_l

    _, acc_s, acc_l = jax.lax.fori_loop(
        0, U, step,
        (m, jnp.full((SEL, 1), -1, jnp.int32), jnp.full((1, SEL), -1, jnp.int32)))
    its_ref[...] = acc_s
    itl_ref[...] = acc_l


def _attn_body(q_ref, k_ref, v_ref, its_ref, delta_ref, vmean_ref):
    ti = its_ref[...]
    onehot = (jax.lax.broadcasted_iota(jnp.int32, (SEL, N), 1) == ti
              ).astype(jnp.float32)
    qr = jnp.dot(onehot, q_ref[...], preferred_element_type=jnp.float32)
    sc = _dot_t(qr, k_ref[...]) * SCALE
    mx = jnp.max(sc, axis=1, keepdims=True)
    e = jnp.exp(sc - mx)
    attn = e / jnp.sum(e, axis=1, keepdims=True)
    upd = jnp.dot(attn, v_ref[...], preferred_element_type=jnp.float32)
    vmean = jnp.mean(v_ref[...], axis=0, keepdims=True)
    delta_ref[...] = upd - vmean
    vmean_ref[...] = vmean


def _final_body(wf_ref, add_ref, delta_ref, vmean_ref, itl_ref, bfin_ref, out_ref):
    t = pl.program_id(0)
    rowi = jax.lax.broadcasted_iota(jnp.int32, (ET, SEL), 0) + t * ET
    scat = (rowi == itl_ref[...]).astype(jnp.float32)
    ctx = add_ref[...] + vmean_ref[...] + jnp.dot(
        scat, delta_ref[...], preferred_element_type=jnp.float32)

    @pl.when(t == 0)
    def _():
        out_ref[...] = bfin_ref[...]

    part = jnp.sum(wf_ref[...] * ctx[None, :, :], axis=(1, 2))
    out_ref[...] += part[None, :]


def kernel(input_embedding, W_lin, b_lin, W_q, W_k, W_v, W_add, b_add, W_fin, b_fin):
    f32 = jnp.float32
    emb = input_embedding.reshape(N, D_IN)
    counts = jnp.asarray(_sample_counts())

    q, k, v, add = pl.pallas_call(
        _proj_body,
        grid=(N // QT,),
        in_specs=[
            pl.BlockSpec((QT, D_IN), lambda i: (i, 0)),
            pl.BlockSpec((D, D_IN), lambda i: (0, 0)),
            pl.BlockSpec((1, D), lambda i: (0, 0)),
            pl.BlockSpec((D, D), lambda i: (0, 0)),
            pl.BlockSpec((D, D), lambda i: (0, 0)),
            pl.BlockSpec((D, D), lambda i: (0, 0)),
            pl.BlockSpec((D, D), lambda i: (0, 0)),
            pl.BlockSpec((1, D), lambda i: (0, 0)),
        ],
        out_specs=[pl.BlockSpec((QT, D), lambda i: (i, 0))] * 4,
        out_shape=[jax.ShapeDtypeStruct((N, D), f32)] * 4,
    )(emb, W_lin, b_lin.reshape(1, D), W_q, W_k, W_v, W_add, b_add.reshape(1, D))

    m = pl.pallas_call(
        _measure_body,
        grid=(N // QT,),
        in_specs=[
            pl.BlockSpec((QT, D), lambda i: (i, 0)),
            pl.BlockSpec((N, D), lambda i: (0, 0)),
            pl.BlockSpec((QT, N), lambda i: (i, 0)),
        ],
        out_specs=pl.BlockSpec((QT, 1), lambda i: (i, 0)),
        out_shape=jax.ShapeDtypeStruct((N, 1), f32),
    )(q, k, counts)

    its, itl = pl.pallas_call(
        _topk_body,
        in_specs=[pl.BlockSpec((16, 128), lambda: (0, 0))],
        out_specs=[pl.BlockSpec((SEL, 1), lambda: (0, 0)),
                   pl.BlockSpec((1, SEL), lambda: (0, 0))],
        out_shape=[jax.ShapeDtypeStruct((SEL, 1), jnp.int32),
                   jax.ShapeDtypeStruct((1, SEL), jnp.int32)],
    )(m.reshape(16, 128))

    delta, vmean = pl.pallas_call(
        _attn_body,
        in_specs=[
            pl.BlockSpec((N, D), lambda: (0, 0)),
            pl.BlockSpec((N, D), lambda: (0, 0)),
            pl.BlockSpec((N, D), lambda: (0, 0)),
            pl.BlockSpec((SEL, 1), lambda: (0, 0)),
        ],
        out_specs=[pl.BlockSpec((SEL, D), lambda: (0, 0)),
                   pl.BlockSpec((1, D), lambda: (0, 0))],
        out_shape=[jax.ShapeDtypeStruct((SEL, D), f32),
                   jax.ShapeDtypeStruct((1, D), f32)],
    )(q, k, v, its)

    out = pl.pallas_call(
        _final_body,
        grid=(N // ET,),
        in_specs=[
            pl.BlockSpec((NCLS, ET, D), lambda t: (0, t, 0)),
            pl.BlockSpec((ET, D), lambda t: (t, 0)),
            pl.BlockSpec((SEL, D), lambda t: (0, 0)),
            pl.BlockSpec((1, D), lambda t: (0, 0)),
            pl.BlockSpec((1, SEL), lambda t: (0, 0)),
            pl.BlockSpec((1, NCLS), lambda t: (0, 0)),
        ],
        out_specs=pl.BlockSpec((1, NCLS), lambda t: (0, 0)),
        out_shape=jax.ShapeDtypeStruct((1, NCLS), f32),
    )(W_fin.reshape(NCLS, N, D), add, delta, vmean, itl, b_fin.reshape(1, NCLS))

    return out


# speedup vs baseline: 3.9706x; 1.0811x over previous
import functools
import math

import numpy as np
import jax
import jax.numpy as jnp
from jax.experimental import pallas as pl
from jax.experimental.pallas import tpu as pltpu

N = 2048
D_IN = 1024
D = 768
NCLS = 16
U = 40
SEL = 64
QT = 256
ET = 128
SCALE = 1.0 / math.sqrt(D)


@functools.lru_cache(maxsize=None)
def _sample_counts():
    with jax.ensure_compile_time_eval():
        idx = np.asarray(jax.random.randint(jax.random.key(42), (N, U), 0, N))
    c = np.zeros((N, N), dtype=np.int8)
    np.add.at(c, (np.arange(N)[:, None], idx.astype(np.int64)), 1)
    return c


def _dot_t(a, b):
    return jax.lax.dot_general(a, b, (((1,), (1,)), ((), ())),
                               preferred_element_type=jnp.float32)


def _proj_body(emb_ref, wl_ref, bl_ref, wq_ref, wk_ref, wv_ref, wa_ref, ba_ref,
               q_ref, k_ref, v_ref, add_ref):
    x = _dot_t(emb_ref[...], wl_ref[...]) + bl_ref[...]
    q_ref[...] = _dot_t(x, wq_ref[...])
    k_ref[...] = _dot_t(x, wk_ref[...])
    v_ref[...] = _dot_t(x, wv_ref[...])
    add_ref[...] = _dot_t(x, wa_ref[...]) + ba_ref[...]


def _measure_body(q_ref, k_ref, c_ref, m_ref):
    s = _dot_t(q_ref[...], k_ref[...])
    cf = c_ref[...].astype(jnp.float32)
    smax = jnp.max(jnp.where(cf > 0.5, s, -1e30), axis=1, keepdims=True)
    ssum = jnp.sum(s * cf, axis=1, keepdims=True)
    m_ref[...] = smax - ssum * (1.0 / N)


def _select_body(m_ref, q_ref, k_ref, v_ref, add_ref, ctx_ref):
    m = m_ref[...]
    rows = jax.lax.broadcasted_iota(jnp.int32, (16, 128), 0)
    lanes = jax.lax.broadcasted_iota(jnp.int32, (16, 128), 1)
    gidx = rows * 128 + lanes
    sel_s = jax.lax.broadcasted_iota(jnp.int32, (SEL, 1), 0)

    def step(j, carry):
        mm, acc_s = carry
        cmax = jnp.max(mm)
        sel = jnp.min(jnp.where(mm == cmax, gidx, jnp.int32(1 << 30)))
        acc_s = jnp.where(sel_s == j, sel, acc_s)
        mm = jnp.where(gidx == sel, -3e38, mm)
        return mm, acc_s

    _, ti = jax.lax.fori_loop(
        0, U, step, (m, jnp.full((SEL, 1), -1, jnp.int32)))

    onehot = (jax.lax.broadcasted_iota(jnp.int32, (SEL, N), 1) == ti
              ).astype(jnp.float32)
    qr = jnp.dot(onehot, q_ref[...], preferred_element_type=jnp.float32)
    sc = _dot_t(qr, k_ref[...]) * SCALE
    mx = jnp.max(sc, axis=1, keepdims=True)
    e = jnp.exp(sc - mx)
    attn = e / jnp.sum(e, axis=1, keepdims=True)
    upd = jnp.dot(attn, v_ref[...], preferred_element_type=jnp.float32)
    vmean = jnp.mean(v_ref[...], axis=0, keepdims=True)
    delta = upd - vmean
    scat = jax.lax.dot_general(onehot, delta, (((0,), (0,)), ((), ())),
                               preferred_element_type=jnp.float32)
    ctx_ref[...] = add_ref[...] + vmean + scat


def _wfin_body(wf_ref, ctxf_ref, bfin_ref, out_ref):
    t = pl.program_id(0)

    @pl.when(t == 0)
    def _():
        out_ref[...] = bfin_ref[...]

    part = jnp.sum(wf_ref[...] * ctxf_ref[...], axis=1)
    out_ref[...] += part[None, :]


def kernel(input_embedding, W_lin, b_lin, W_q, W_k, W_v, W_add, b_add, W_fin, b_fin):
    f32 = jnp.float32
    emb = input_embedding.reshape(N, D_IN)
    counts = jnp.asarray(_sample_counts())

    q, k, v, add = pl.pallas_call(
        _proj_body,
        grid=(N // QT,),
        in_specs=[
            pl.BlockSpec((QT, D_IN), lambda i: (i, 0)),
            pl.BlockSpec((D, D_IN), lambda i: (0, 0)),
            pl.BlockSpec((1, D), lambda i: (0, 0)),
            pl.BlockSpec((D, D), lambda i: (0, 0)),
            pl.BlockSpec((D, D), lambda i: (0, 0)),
            pl.BlockSpec((D, D), lambda i: (0, 0)),
            pl.BlockSpec((D, D), lambda i: (0, 0)),
            pl.BlockSpec((1, D), lambda i: (0, 0)),
        ],
        out_specs=[pl.BlockSpec((QT, D), lambda i: (i, 0))] * 4,
        out_shape=[jax.ShapeDtypeStruct((N, D), f32)] * 4,
    )(emb, W_lin, b_lin.reshape(1, D), W_q, W_k, W_v, W_add, b_add.reshape(1, D))

    m = pl.pallas_call(
        _measure_body,
        grid=(N // QT,),
        in_specs=[
            pl.BlockSpec((QT, D), lambda i: (i, 0)),
            pl.BlockSpec((N, D), lambda i: (0, 0)),
            pl.BlockSpec((QT, N), lambda i: (i, 0)),
        ],
        out_specs=pl.BlockSpec((QT, 1), lambda i: (i, 0)),
        out_shape=jax.ShapeDtypeStruct((N, 1), f32),
    )(q, k, counts)

    ctx = pl.pallas_call(
        _select_body,
        in_specs=[
            pl.BlockSpec((16, 128), lambda: (0, 0)),
            pl.BlockSpec((N, D), lambda: (0, 0)),
            pl.BlockSpec((N, D), lambda: (0, 0)),
            pl.BlockSpec((N, D), lambda: (0, 0)),
            pl.BlockSpec((N, D), lambda: (0, 0)),
        ],
        out_specs=pl.BlockSpec((N, D), lambda: (0, 0)),
        out_shape=jax.ShapeDtypeStruct((N, D), f32),
    )(m.reshape(16, 128), q, k, v, add)

    ctxf = ctx.reshape(1, N * D)
    CH = (N * D) // 32
    out = pl.pallas_call(
        _wfin_body,
        grid=(32,),
        in_specs=[
            pl.BlockSpec((NCLS, CH), lambda t: (0, t)),
            pl.BlockSpec((1, CH), lambda t: (0, t)),
            pl.BlockSpec((1, NCLS), lambda t: (0, 0)),
        ],
        out_specs=pl.BlockSpec((1, NCLS), lambda t: (0, 0)),
        out_shape=jax.ShapeDtypeStruct((1, NCLS), f32),
    )(W_fin, ctxf, b_fin.reshape(1, NCLS))

    return out


# speedup vs baseline: 4.6111x; 1.1613x over previous
import functools
import math

import numpy as np
import jax
import jax.numpy as jnp
from jax.experimental import pallas as pl
from jax.experimental.pallas import tpu as pltpu

N = 2048
D_IN = 1024
D = 768
NCLS = 16
U = 40
SEL = 64
QT = 256
ET = 128
SCALE = 1.0 / math.sqrt(D)


def _rotl(x, d):
    return ((x << np.uint32(d)) | (x >> np.uint32(32 - d))).astype(np.uint32)


def _tf2x32(k0, k1, x0, x1):
    rot = [[13, 15, 26, 6], [17, 29, 16, 24]]
    k0 = np.uint32(k0)
    k1 = np.uint32(k1)
    ks = [k0, k1, np.uint32(k0 ^ k1 ^ np.uint32(0x1BD11BDA))]
    x0 = (x0 + ks[0]).astype(np.uint32)
    x1 = (x1 + ks[1]).astype(np.uint32)
    for i in range(5):
        for r in rot[i % 2]:
            x0 = (x0 + x1).astype(np.uint32)
            x1 = _rotl(x1, r)
            x1 = (x1 ^ x0).astype(np.uint32)
        x0 = (x0 + ks[(i + 1) % 3]).astype(np.uint32)
        x1 = (x1 + ks[(i + 2) % 3] + np.uint32(i + 1)).astype(np.uint32)
    return x0, x1


def _bits32(k, n):
    b1, b2 = _tf2x32(k[0], k[1], np.zeros(n, np.uint32),
                     np.arange(n, dtype=np.uint32))
    return (b1 ^ b2).astype(np.uint32)


@functools.lru_cache(maxsize=None)
def _sample_counts():
    b1, b2 = _tf2x32(0, 42, np.zeros(2, np.uint32), np.arange(2, dtype=np.uint32))
    k1, k2 = (b1[0], b2[0]), (b1[1], b2[1])
    n = N * U
    hi, lo = _bits32(k1, n), _bits32(k2, n)
    span = np.uint32(N)
    mult = np.uint32((((2 ** 16) % N) ** 2) % N)
    idx = (((hi % span) * mult + (lo % span)) % span).astype(np.int64).reshape(N, U)
    c = np.zeros((N, N), dtype=np.int8)
    np.add.at(c, (np.arange(N)[:, None], idx), 1)
    return c


def _dot_t(a, b):
    return jax.lax.dot_general(a, b, (((1,), (1,)), ((), ())),
                               preferred_element_type=jnp.float32)


def _main_body(emb_ref, wl_ref, bl_ref, wq_ref, wk_ref, wv_ref, wa_ref, ba_ref,
               c_ref, ctx_ref, q_sc, k_sc, v_sc, m_sc):
    NT = N // QT
    for t in range(NT):
        sl = pl.ds(t * QT, QT)
        x = _dot_t(emb_ref[sl, :], wl_ref[...]) + bl_ref[...]
        q_sc[sl, :] = _dot_t(x, wq_ref[...])
        k_sc[sl, :] = _dot_t(x, wk_ref[...])
        v_sc[sl, :] = _dot_t(x, wv_ref[...])
        ctx_ref[sl, :] = _dot_t(x, wa_ref[...]) + ba_ref[...]

    for t in range(NT):
        sl = pl.ds(t * QT, QT)
        s = _dot_t(q_sc[sl, :], k_sc[...])
        cf = c_ref[sl, :].astype(jnp.float32)
        smax = jnp.max(jnp.where(cf > 0.5, s, -1e30), axis=1, keepdims=True)
        ssum = jnp.sum(s * cf, axis=1, keepdims=True)
        m_sc[:, pl.ds(t, 1)] = smax - ssum * (1.0 / N)

    m = m_sc[...]
    rows = jax.lax.broadcasted_iota(jnp.int32, (QT, NT), 0)
    lanes = jax.lax.broadcasted_iota(jnp.int32, (QT, NT), 1)
    gidx = lanes * QT + rows
    sel_s = jax.lax.broadcasted_iota(jnp.int32, (SEL, 1), 0)

    def step(j, carry):
        mm, acc_s = carry
        cmax = jnp.max(mm)
        sel = jnp.min(jnp.where(mm == cmax, gidx, jnp.int32(1 << 30)))
        acc_s = jnp.where(sel_s == j, sel, acc_s)
        mm = jnp.where(gidx == sel, -3e38, mm)
        return mm, acc_s

    _, ti = jax.lax.fori_loop(
        0, U, step, (m, jnp.full((SEL, 1), -1, jnp.int32)))

    onehot = (jax.lax.broadcasted_iota(jnp.int32, (SEL, N), 1) == ti
              ).astype(jnp.float32)
    qr = jnp.dot(onehot, q_sc[...], preferred_element_type=jnp.float32)
    sc = _dot_t(qr, k_sc[...]) * SCALE
    mx = jnp.max(sc, axis=1, keepdims=True)
    e = jnp.exp(sc - mx)
    attn = e / jnp.sum(e, axis=1, keepdims=True)
    upd = jnp.dot(attn, v_sc[...], preferred_element_type=jnp.float32)
    vmean = jnp.mean(v_sc[...], axis=0, keepdims=True)
    delta = upd - vmean
    for t in range(NT):
        sl = pl.ds(t * QT, QT)
        oh_t = onehot[:, t * QT:(t + 1) * QT]
        scat = jax.lax.dot_general(oh_t, delta, (((0,), (0,)), ((), ())),
                                   preferred_element_type=jnp.float32)
        ctx_ref[sl, :] = ctx_ref[sl, :] + vmean + scat


def _wfin_body(wf_ref, ctxf_ref, bfin_ref, out_ref):
    t = pl.program_id(0)

    @pl.when(t == 0)
    def _():
        out_ref[...] = bfin_ref[...]

    part = jnp.sum(wf_ref[...] * ctxf_ref[...], axis=1)
    out_ref[...] += part[None, :]


def kernel(input_embedding, W_lin, b_lin, W_q, W_k, W_v, W_add, b_add, W_fin, b_fin):
    f32 = jnp.float32
    emb = input_embedding.reshape(N, D_IN)
    counts = jnp.asarray(_sample_counts())

    ctx = pl.pallas_call(
        _main_body,
        in_specs=[
            pl.BlockSpec((N, D_IN), lambda: (0, 0)),
            pl.BlockSpec((D, D_IN), lambda: (0, 0)),
            pl.BlockSpec((1, D), lambda: (0, 0)),
            pl.BlockSpec((D, D), lambda: (0, 0)),
            pl.BlockSpec((D, D), lambda: (0, 0)),
            pl.BlockSpec((D, D), lambda: (0, 0)),
            pl.BlockSpec((D, D), lambda: (0, 0)),
            pl.BlockSpec((1, D), lambda: (0, 0)),
            pl.BlockSpec((N, N), lambda: (0, 0)),
        ],
        out_specs=pl.BlockSpec((N, D), lambda: (0, 0)),
        out_shape=jax.ShapeDtypeStruct((N, D), f32),
        scratch_shapes=[
            pltpu.VMEM((N, D), f32),
            pltpu.VMEM((N, D), f32),
            pltpu.VMEM((N, D), f32),
            pltpu.VMEM((QT, N // QT), f32),
        ],
    )(emb, W_lin, b_lin.reshape(1, D), W_q, W_k, W_v, W_add,
      b_add.reshape(1, D), counts)

    ctxf = ctx.reshape(1, N * D)
    CH = (N * D) // 32
    out = pl.pallas_call(
        _wfin_body,
        grid=(32,),
        in_specs=[
            pl.BlockSpec((NCLS, CH), lambda t: (0, t)),
            pl.BlockSpec((1, CH), lambda t: (0, t)),
            pl.BlockSpec((1, NCLS), lambda t: (0, 0)),
        ],
        out_specs=pl.BlockSpec((1, NCLS), lambda t: (0, 0)),
        out_shape=jax.ShapeDtypeStruct((1, NCLS), f32),
    )(W_fin, ctxf, b_fin.reshape(1, NCLS))

    return out


# speedup vs baseline: 4.7252x; 1.0247x over previous
import functools
import math

import numpy as np
import jax
from jax import lax
import jax.numpy as jnp
from jax.experimental import pallas as pl
from jax.experimental.pallas import tpu as pltpu
from jax.experimental.pallas import tpu_sc as plsc

N = 2048
D_IN = 1024
D = 768
NCLS = 16
U = 40
SEL = 64
QT = 256
ET = 128
SCALE = 1.0 / math.sqrt(D)


def _rotl(x, d):
    return ((x << np.uint32(d)) | (x >> np.uint32(32 - d))).astype(np.uint32)


def _tf2x32(k0, k1, x0, x1):
    rot = [[13, 15, 26, 6], [17, 29, 16, 24]]
    k0 = np.uint32(k0)
    k1 = np.uint32(k1)
    ks = [k0, k1, np.uint32(k0 ^ k1 ^ np.uint32(0x1BD11BDA))]
    x0 = (x0 + ks[0]).astype(np.uint32)
    x1 = (x1 + ks[1]).astype(np.uint32)
    for i in range(5):
        for r in rot[i % 2]:
            x0 = (x0 + x1).astype(np.uint32)
            x1 = _rotl(x1, r)
            x1 = (x1 ^ x0).astype(np.uint32)
        x0 = (x0 + ks[(i + 1) % 3]).astype(np.uint32)
        x1 = (x1 + ks[(i + 2) % 3] + np.uint32(i + 1)).astype(np.uint32)
    return x0, x1


def _bits32(k, n):
    b1, b2 = _tf2x32(k[0], k[1], np.zeros(n, np.uint32),
                     np.arange(n, dtype=np.uint32))
    return (b1 ^ b2).astype(np.uint32)


@functools.lru_cache(maxsize=None)
def _sample_counts():
    b1, b2 = _tf2x32(0, 42, np.zeros(2, np.uint32), np.arange(2, dtype=np.uint32))
    k1, k2 = (b1[0], b2[0]), (b1[1], b2[1])
    n = N * U
    hi, lo = _bits32(k1, n), _bits32(k2, n)
    span = np.uint32(N)
    mult = np.uint32((((2 ** 16) % N) ** 2) % N)
    idx = (((hi % span) * mult + (lo % span)) % span).astype(np.int64).reshape(N, U)
    c = np.zeros((N, N), dtype=np.int8)
    np.add.at(c, (np.arange(N)[:, None], idx), 1)
    return c


def _dot_t(a, b):
    return jax.lax.dot_general(a, b, (((1,), (1,)), ((), ())),
                               preferred_element_type=jnp.float32)


def _main_body(emb_ref, wl_ref, bl_ref, wq_ref, wk_ref, wv_ref, wa_ref, ba_ref,
               c_ref, ctx_ref, q_sc, k_sc, v_sc, m_sc):
    NT = N // QT
    for t in range(NT):
        sl = pl.ds(t * QT, QT)
        x = _dot_t(emb_ref[sl, :], wl_ref[...]) + bl_ref[...]
        q_sc[sl, :] = _dot_t(x, wq_ref[...])
        k_sc[sl, :] = _dot_t(x, wk_ref[...])
        v_sc[sl, :] = _dot_t(x, wv_ref[...])
        ctx_ref[sl, :] = _dot_t(x, wa_ref[...]) + ba_ref[...]

    for t in range(NT):
        sl = pl.ds(t * QT, QT)
        s = _dot_t(q_sc[sl, :], k_sc[...])
        cf = c_ref[sl, :].astype(jnp.float32)
        smax = jnp.max(jnp.where(cf > 0.5, s, -1e30), axis=1, keepdims=True)
        ssum = jnp.sum(s * cf, axis=1, keepdims=True)
        m_sc[:, pl.ds(t, 1)] = smax - ssum * (1.0 / N)

    m = m_sc[...]
    rows = jax.lax.broadcasted_iota(jnp.int32, (QT, NT), 0)
    lanes = jax.lax.broadcasted_iota(jnp.int32, (QT, NT), 1)
    gidx = lanes * QT + rows
    sel_s = jax.lax.broadcasted_iota(jnp.int32, (SEL, 1), 0)

    def step(j, carry):
        mm, acc_s = carry
        cmax = jnp.max(mm)
        sel = jnp.min(jnp.where(mm == cmax, gidx, jnp.int32(1 << 30)))
        acc_s = jnp.where(sel_s == j, sel, acc_s)
        mm = jnp.where(gidx == sel, -3e38, mm)
        return mm, acc_s

    _, ti = jax.lax.fori_loop(
        0, U, step, (m, jnp.full((SEL, 1), -1, jnp.int32)))

    onehot = (jax.lax.broadcasted_iota(jnp.int32, (SEL, N), 1) == ti
              ).astype(jnp.float32)
    qr = jnp.dot(onehot, q_sc[...], preferred_element_type=jnp.float32)
    sc = _dot_t(qr, k_sc[...]) * SCALE
    mx = jnp.max(sc, axis=1, keepdims=True)
    e = jnp.exp(sc - mx)
    attn = e / jnp.sum(e, axis=1, keepdims=True)
    upd = jnp.dot(attn, v_sc[...], preferred_element_type=jnp.float32)
    vmean = jnp.mean(v_sc[...], axis=0, keepdims=True)
    delta = upd - vmean
    for t in range(NT):
        sl = pl.ds(t * QT, QT)
        oh_t = onehot[:, t * QT:(t + 1) * QT]
        scat = jax.lax.dot_general(oh_t, delta, (((0,), (0,)), ((), ())),
                                   preferred_element_type=jnp.float32)
        ctx_ref[sl, :] = ctx_ref[sl, :] + vmean + scat


NW = 32
SC_PER_W = 12288
SC_F = NW * SC_PER_W
SC_CHUNK = 3072


def _sc_tail_body(wf_hbm, ctxf_hbm, out_hbm, wbuf, cbuf, obuf):
    wid = lax.axis_index("s") * 2 + lax.axis_index("c")
    base = (N * D - SC_F) + wid * SC_PER_W
    accs = tuple(jnp.zeros((16,), jnp.float32) for _ in range(NCLS))
    for ch in range(SC_PER_W // SC_CHUNK):
        off = base + ch * SC_CHUNK
        pltpu.sync_copy(wf_hbm.at[:, pl.ds(off, SC_CHUNK)], wbuf)
        pltpu.sync_copy(ctxf_hbm.at[0, pl.ds(off, SC_CHUNK)], cbuf)

        def body(i, a):
            cvec = cbuf[pl.ds(i * 16, 16)]
            return tuple(a[c] + wbuf[c, pl.ds(i * 16, 16)] * cvec
                         for c in range(NCLS))

        accs = lax.fori_loop(0, SC_CHUNK // 16, body, accs)
    for c in range(NCLS):
        obuf[c, :] = accs[c]
    pltpu.sync_copy(obuf, out_hbm.at[wid])


def _sc_tail(wf, ctxf):
    mesh = plsc.VectorSubcoreMesh(core_axis_name="c", subcore_axis_name="s")
    fn = functools.partial(
        pl.kernel, mesh=mesh,
        out_type=jax.ShapeDtypeStruct((NW, NCLS, 16), jnp.float32),
        scratch_types=[
            pltpu.VMEM((NCLS, SC_CHUNK), jnp.float32),
            pltpu.VMEM((SC_CHUNK,), jnp.float32),
            pltpu.VMEM((NCLS, 16), jnp.float32),
        ],
    )(_sc_tail_body)
    return fn(wf, ctxf)


def _wfin_body(wf_ref, ctxf_ref, bfin_ref, out_ref):
    t = pl.program_id(0)

    @pl.when(t == 0)
    def _():
        out_ref[...] = bfin_ref[...]

    part = jnp.sum(wf_ref[...] * ctxf_ref[...], axis=1)
    out_ref[...] += part[None, :]


def kernel(input_embedding, W_lin, b_lin, W_q, W_k, W_v, W_add, b_add, W_fin, b_fin):
    f32 = jnp.float32
    emb = input_embedding.reshape(N, D_IN)
    counts = jnp.asarray(_sample_counts())

    ctx = pl.pallas_call(
        _main_body,
        in_specs=[
            pl.BlockSpec((N, D_IN), lambda: (0, 0)),
            pl.BlockSpec((D, D_IN), lambda: (0, 0)),
            pl.BlockSpec((1, D), lambda: (0, 0)),
            pl.BlockSpec((D, D), lambda: (0, 0)),
            pl.BlockSpec((D, D), lambda: (0, 0)),
            pl.BlockSpec((D, D), lambda: (0, 0)),
            pl.BlockSpec((D, D), lambda: (0, 0)),
            pl.BlockSpec((1, D), lambda: (0, 0)),
            pl.BlockSpec((N, N), lambda: (0, 0)),
        ],
        out_specs=pl.BlockSpec((N, D), lambda: (0, 0)),
        out_shape=jax.ShapeDtypeStruct((N, D), f32),
        scratch_shapes=[
            pltpu.VMEM((N, D), f32),
            pltpu.VMEM((N, D), f32),
            pltpu.VMEM((N, D), f32),
            pltpu.VMEM((QT, N // QT), f32),
        ],
    )(emb, W_lin, b_lin.reshape(1, D), W_q, W_k, W_v, W_add,
      b_add.reshape(1, D), counts)

    ctxf = ctx.reshape(1, N * D)
    out_sc = _sc_tail(W_fin, ctxf)

    CH = (N * D) // 32
    NSTEP = (N * D - SC_F) // CH
    out_tc = pl.pallas_call(
        _wfin_body,
        grid=(NSTEP,),
        in_specs=[
            pl.BlockSpec((NCLS, CH), lambda t: (0, t)),
            pl.BlockSpec((1, CH), lambda t: (0, t)),
            pl.BlockSpec((1, NCLS), lambda t: (0, 0)),
        ],
        out_specs=pl.BlockSpec((1, NCLS), lambda t: (0, 0)),
        out_shape=jax.ShapeDtypeStruct((1, NCLS), f32),
    )(W_fin, ctxf, b_fin.reshape(1, NCLS))

    return out_tc + jnp.sum(out_sc, axis=(0, 2))[None, :]
